# baseline (speedup 1.0000x reference)
import functools

import jax
import jax.numpy as jnp
from jax.experimental import pallas as pl
from jax.experimental.pallas import tpu as pltpu

E = 64


def _router_body(x_ref, w_hid_ref, b_hid_ref, emb_ref, w_exp_ref, b_exp_ref,
                 logits_ref, idx_ref, w_ref, whid_bf_ref, emb_bf_ref):
    step = pl.program_id(0)

    @pl.when(step == 0)
    def _prologue():
        whid_bf_ref[...] = w_hid_ref[...].astype(jnp.bfloat16)
        rkhs_emb = jax.lax.dot_general(
            emb_ref[...].astype(jnp.bfloat16),
            w_exp_ref[...].astype(jnp.bfloat16),
            (((1,), (1,)), ((), ())),
            preferred_element_type=jnp.float32) + b_exp_ref[...]
        emb_bf_ref[...] = rkhs_emb.astype(jnp.bfloat16)

    enc = jax.lax.dot_general(
        x_ref[...].astype(jnp.bfloat16), whid_bf_ref[...],
        (((1,), (1,)), ((), ())),
        preferred_element_type=jnp.float32) + b_hid_ref[...]
    logits = jax.lax.dot_general(
        enc.astype(jnp.bfloat16), emb_bf_ref[...],
        (((1,), (1,)), ((), ())),
        preferred_element_type=jnp.float32)
    logits_ref[...] = logits

    prev = logits
    iota = jax.lax.broadcasted_iota(jnp.int32, prev.shape, 1).astype(jnp.float32)
    m1 = jnp.max(prev, axis=1, keepdims=True)
    i1 = jnp.min(jnp.where(prev == m1, iota, float(E)), axis=1, keepdims=True)
    masked = jnp.where(iota == i1, -jnp.inf, prev)
    m2 = jnp.max(masked, axis=1, keepdims=True)
    i2 = jnp.min(jnp.where(masked == m2, iota, float(E)), axis=1, keepdims=True)
    idx_ref[...] = jnp.concatenate([i1, i2], axis=1).astype(jnp.int32)

    e2 = jnp.exp(m2 - m1)
    denom = 1.0 + e2
    w_ref[...] = jnp.concatenate([1.0 / denom, e2 / denom], axis=1)


@functools.partial(jax.jit, static_argnames=("bm",))
def _router(x, W_hid, b_hid_row, emb, W_exp, b_exp_row, bm):
    M, D = x.shape
    RKHS = W_hid.shape[0]
    grid = (M // bm,)
    logits, idx, topw = pl.pallas_call(
        _router_body,
        grid=grid,
        in_specs=[
            pl.BlockSpec((bm, D), lambda i: (i, 0)),
            pl.BlockSpec((RKHS, D), lambda i: (0, 0)),
            pl.BlockSpec((1, RKHS), lambda i: (0, 0)),
            pl.BlockSpec((E, W_exp.shape[1]), lambda i: (0, 0)),
            pl.BlockSpec((RKHS, W_exp.shape[1]), lambda i: (0, 0)),
            pl.BlockSpec((1, RKHS), lambda i: (0, 0)),
        ],
        out_specs=[
            pl.BlockSpec((bm, E), lambda i: (i, 0)),
            pl.BlockSpec((bm, 2), lambda i: (i, 0)),
            pl.BlockSpec((bm, 2), lambda i: (i, 0)),
        ],
        out_shape=[
            jax.ShapeDtypeStruct((M, E), jnp.float32),
            jax.ShapeDtypeStruct((M, 2), jnp.int32),
            jax.ShapeDtypeStruct((M, 2), jnp.float32),
        ],
        scratch_shapes=[
            pltpu.VMEM((RKHS, D), jnp.bfloat16),
            pltpu.VMEM((E, RKHS), jnp.bfloat16),
        ],
        compiler_params=pltpu.CompilerParams(
            dimension_semantics=("arbitrary",),
        ),
    )(x, W_hid, b_hid_row, emb, W_exp, b_exp_row)
    return logits, idx, topw


def kernel(input, W_hid, b_hid, W_exp, b_exp, rkhs_embeddings):
    B, S, D = input.shape
    x = input.reshape(B * S, D)
    logits, idx, topw = _router(
        x, W_hid, b_hid.reshape(1, -1), rkhs_embeddings, W_exp,
        b_exp.reshape(1, -1), bm=1024)
    return (idx.reshape(B, S, 2),
            topw.astype(input.dtype).reshape(B, S, 2),
            logits.reshape(B, S, E))


# speedup vs baseline: 1.0044x; 1.0044x over previous
import functools

import jax
import jax.numpy as jnp
from jax.experimental import pallas as pl
from jax.experimental.pallas import tpu as pltpu

E = 64


def _router_body(x_ref, w_hid_ref, b_hid_ref, emb_ref, w_exp_ref, b_exp_ref,
                 logits_ref, idx_ref, w_ref, whid_bf_ref, emb_bf_ref):
    step = pl.program_id(0)

    @pl.when(step == 0)
    def _prologue():
        whid_bf_ref[...] = w_hid_ref[...].astype(jnp.bfloat16)
        rkhs_emb = jax.lax.dot_general(
            emb_ref[...].astype(jnp.bfloat16),
            w_exp_ref[...].astype(jnp.bfloat16),
            (((1,), (1,)), ((), ())),
            preferred_element_type=jnp.float32) + b_exp_ref[...]
        emb_bf_ref[...] = rkhs_emb.astype(jnp.bfloat16)

    enc = jax.lax.dot_general(
        x_ref[...], whid_bf_ref[...],
        (((1,), (1,)), ((), ())),
        precision=jax.lax.Precision.DEFAULT,
        preferred_element_type=jnp.float32) + b_hid_ref[...]
    logits = jax.lax.dot_general(
        enc.astype(jnp.bfloat16), emb_bf_ref[...],
        (((1,), (1,)), ((), ())),
        preferred_element_type=jnp.float32)
    logits_ref[...] = logits

    prev = logits
    iota = jax.lax.broadcasted_iota(jnp.int32, prev.shape, 1).astype(jnp.float32)
    m1 = jnp.max(prev, axis=1, keepdims=True)
    i1 = jnp.min(jnp.where(prev == m1, iota, float(E)), axis=1, keepdims=True)
    masked = jnp.where(iota == i1, -jnp.inf, prev)
    m2 = jnp.max(masked, axis=1, keepdims=True)
    i2 = jnp.min(jnp.where(masked == m2, iota, float(E)), axis=1, keepdims=True)
    idx_ref[...] = jnp.concatenate([i1, i2], axis=1).astype(jnp.int32)

    e2 = jnp.exp(m2 - m1)
    denom = 1.0 + e2
    w_ref[...] = jnp.concatenate([1.0 / denom, e2 / denom], axis=1)


@functools.partial(jax.jit, static_argnames=("bm",))
def _router(x, W_hid, b_hid_row, emb, W_exp, b_exp_row, bm):
    M, D = x.shape
    RKHS = W_hid.shape[0]
    grid = (M // bm,)
    logits, idx, topw = pl.pallas_call(
        _router_body,
        grid=grid,
        in_specs=[
            pl.BlockSpec((bm, D), lambda i: (i, 0)),
            pl.BlockSpec((RKHS, D), lambda i: (0, 0)),
            pl.BlockSpec((1, RKHS), lambda i: (0, 0)),
            pl.BlockSpec((E, W_exp.shape[1]), lambda i: (0, 0)),
            pl.BlockSpec((RKHS, W_exp.shape[1]), lambda i: (0, 0)),
            pl.BlockSpec((1, RKHS), lambda i: (0, 0)),
        ],
        out_specs=[
            pl.BlockSpec((bm, E), lambda i: (i, 0)),
            pl.BlockSpec((bm, 2), lambda i: (i, 0)),
            pl.BlockSpec((bm, 2), lambda i: (i, 0)),
        ],
        out_shape=[
            jax.ShapeDtypeStruct((M, E), jnp.float32),
            jax.ShapeDtypeStruct((M, 2), jnp.int32),
            jax.ShapeDtypeStruct((M, 2), jnp.float32),
        ],
        scratch_shapes=[
            pltpu.VMEM((RKHS, D), jnp.bfloat16),
            pltpu.VMEM((E, RKHS), jnp.bfloat16),
        ],
        compiler_params=pltpu.CompilerParams(
            dimension_semantics=("arbitrary",),
        ),
    )(x, W_hid, b_hid_row, emb, W_exp, b_exp_row)
    return logits, idx, topw


def kernel(input, W_hid, b_hid, W_exp, b_exp, rkhs_embeddings):
    B, S, D = input.shape
    x = input.reshape(B * S, D)
    logits, idx, topw = _router(
        x, W_hid, b_hid.reshape(1, -1), rkhs_embeddings, W_exp,
        b_exp.reshape(1, -1), bm=1024)
    return (idx.reshape(B, S, 2),
            topw.astype(input.dtype).reshape(B, S, 2),
            logits.reshape(B, S, E))
